# baseline (speedup 1.0000x reference)
import functools

import jax
import jax.numpy as jnp
from jax import lax
from jax.experimental import pallas as pl
from jax.experimental.pallas import tpu as pltpu
from jax.experimental.pallas import tpu_sc as plsc

_N = 10000
_E = 320000
_D = 128
_NUM_HOPS = 3
_NOISE_SCALE = 0.1

_NC = 2
_NS = 16
_NW = _NC * _NS
_CHUNK = 128
_EPT = _E // _NW
_CHUNKS = 2 * (-(-_EPT // (2 * _CHUNK)))
_EPT_PAD = _CHUNKS * _CHUNK
_N_PAD = 10112
_RPT = _N_PAD // _NS

_ISLAB = _CHUNKS // 2

_BLK = 1000


def _sc_agg(h, srcp, dstp, init):
    mesh = plsc.VectorSubcoreMesh(core_axis_name="c", subcore_axis_name="s")

    @functools.partial(
        pl.kernel,
        out_type=jax.ShapeDtypeStruct((_NC, _N_PAD, _D), jnp.float32),
        mesh=mesh,
        scratch_types=[
            pltpu.VMEM_SHARED((_N_PAD, _D), jnp.float32),
            pltpu.VMEM((_ISLAB, _CHUNK), jnp.int32),
            pltpu.VMEM((_ISLAB, _CHUNK), jnp.int32),
            pltpu.VMEM((_CHUNK, _D), jnp.float32),
            pltpu.VMEM((_CHUNK, _D), jnp.float32),
            pltpu.SemaphoreType.DMA,
            pltpu.SemaphoreType.DMA,
        ],
    )
    def body(h_hbm, src_hbm, dst_hbm, init_hbm, out_hbm, acc, src_v, dst_v,
             rows0, rows1, sem0, sem1):
        c = lax.axis_index("c")
        s = lax.axis_index("s")
        w = s * _NC + c
        base = s * _RPT
        pltpu.sync_copy(init_hbm.at[c, pl.ds(base, _RPT)], acc.at[pl.ds(base, _RPT)])
        plsc.subcore_barrier()

        for slab in range(_CHUNKS // _ISLAB):
            pltpu.sync_copy(src_hbm.at[w, pl.ds(slab * _ISLAB, _ISLAB)], src_v)
            pltpu.sync_copy(dst_hbm.at[w, pl.ds(slab * _ISLAB, _ISLAB)], dst_v)
            def chunk_body(j, carry):
                pltpu.async_copy(h_hbm.at[src_v.at[j]], rows0, sem0).wait()
                pltpu.sync_copy(rows0, acc.at[dst_v.at[j]], add=True)
                return carry

            lax.fori_loop(0, _ISLAB, chunk_body, 0)
        plsc.subcore_barrier()
        pltpu.sync_copy(acc.at[pl.ds(base, _RPT)], out_hbm.at[c, pl.ds(base, _RPT)])

    return body(h, srcp, dstp, init)


def _norm_rows(y):
    norm = jnp.sqrt(jnp.sum(y * y, axis=1, keepdims=True))
    return y / jnp.maximum(norm, 1e-12)


def _tc_norm(x):
    def body(x_ref, o_ref):
        o_ref[...] = _norm_rows(x_ref[...])

    return pl.pallas_call(
        body,
        grid=(_N // _BLK,),
        in_specs=[pl.BlockSpec((_BLK, _D), lambda i: (i, 0))],
        out_specs=pl.BlockSpec((_BLK, _D), lambda i: (i, 0)),
        out_shape=jax.ShapeDtypeStruct((_N, _D), jnp.float32),
    )(x)


def _tc_combine(p):
    def body(p_ref, o_ref):
        o_ref[...] = _norm_rows(p_ref[0] + p_ref[1])

    return pl.pallas_call(
        body,
        grid=(_N // _BLK,),
        in_specs=[pl.BlockSpec((_NC, _BLK, _D), lambda i: (0, i, 0))],
        out_specs=pl.BlockSpec((_BLK, _D), lambda i: (i, 0)),
        out_shape=jax.ShapeDtypeStruct((_N, _D), jnp.float32),
    )(p)


def kernel(x, edge_index):
    dst = edge_index[0]
    src = edge_index[1]
    pad = _EPT_PAD - _EPT
    srcp = jnp.pad(src.reshape(_NW, _EPT), ((0, 0), (0, pad))).reshape(
        _NW, _CHUNKS, _CHUNK)
    dummy = _N + ((jnp.arange(pad, dtype=jnp.int32)[None, :]
                   + 7 * jnp.arange(_NW, dtype=jnp.int32)[:, None])
                  % (_N_PAD - _N))
    dstp = jnp.concatenate([dst.reshape(_NW, _EPT), dummy], axis=1).reshape(
        _NW, _CHUNKS, _CHUNK)

    noise_key = jax.random.key(42)
    zeros_half = jnp.zeros((1, _N_PAD, _D), jnp.float32)
    outs = [_tc_norm(x)]
    for k in range(_NUM_HOPS):
        noise = jax.random.normal(
            jax.random.fold_in(noise_key, k), (_N, _D), jnp.float32) * _NOISE_SCALE
        init = jnp.concatenate(
            [jnp.pad(noise, ((0, _N_PAD - _N), (0, 0)))[None], zeros_half], axis=0)
        p = _sc_agg(outs[k], srcp, dstp, init)
        outs.append(_tc_combine(p[:, :_N]))
    return jnp.stack(outs)


# speedup vs baseline: 1.1358x; 1.1358x over previous
import functools

import jax
import jax.numpy as jnp
from jax import lax
from jax.experimental import pallas as pl
from jax.experimental.pallas import tpu as pltpu
from jax.experimental.pallas import tpu_sc as plsc

_N = 10000
_E = 320000
_D = 128
_NUM_HOPS = 3
_NOISE_SCALE = 0.1

_NC = 2
_NS = 16
_NW = _NC * _NS
_CHUNK = 128
_EPT = _E // _NW
_CHUNKS = 2 * (-(-_EPT // (2 * _CHUNK)))
_EPT_PAD = _CHUNKS * _CHUNK
_N_PAD = 10112
_RPT = _N_PAD // _NS

_ISLAB = _CHUNKS // 2

_BLK = _N_PAD // 16


def _sc_agg(h, srcp, dstp, init):
    mesh = plsc.VectorSubcoreMesh(core_axis_name="c", subcore_axis_name="s")

    @functools.partial(
        pl.kernel,
        out_type=jax.ShapeDtypeStruct((_NC, _N_PAD, _D), jnp.float32),
        mesh=mesh,
        scratch_types=[
            pltpu.VMEM_SHARED((_N_PAD, _D), jnp.float32),
            pltpu.VMEM((_ISLAB, _CHUNK), jnp.int32),
            pltpu.VMEM((_ISLAB, _CHUNK), jnp.int32),
            pltpu.VMEM((_CHUNK, _D), jnp.float32),
            pltpu.VMEM((_CHUNK, _D), jnp.float32),
            pltpu.SemaphoreType.DMA,
            pltpu.SemaphoreType.DMA,
        ],
    )
    def body(h_hbm, src_hbm, dst_hbm, init_hbm, out_hbm, acc, src_v, dst_v,
             rows0, rows1, sem0, sem1):
        c = lax.axis_index("c")
        s = lax.axis_index("s")
        w = s * _NC + c
        base = s * _RPT
        pltpu.sync_copy(init_hbm.at[c, pl.ds(base, _RPT)], acc.at[pl.ds(base, _RPT)])
        plsc.subcore_barrier()

        for slab in range(_CHUNKS // _ISLAB):
            pltpu.sync_copy(src_hbm.at[w, pl.ds(slab * _ISLAB, _ISLAB)], src_v)
            pltpu.sync_copy(dst_hbm.at[w, pl.ds(slab * _ISLAB, _ISLAB)], dst_v)
            pltpu.async_copy(h_hbm.at[src_v.at[0]], rows0, sem0)

            def chunk_body(i, carry):
                j = 2 * i
                pltpu.async_copy(h_hbm.at[src_v.at[j + 1]], rows1, sem1)
                pltpu.make_async_copy(h_hbm.at[src_v.at[j]], rows0, sem0).wait()
                pltpu.sync_copy(rows0, acc.at[dst_v.at[j]], add=True)
                pltpu.async_copy(h_hbm.at[src_v.at[j + 2]], rows0, sem0)
                pltpu.make_async_copy(h_hbm.at[src_v.at[j + 1]], rows1, sem1).wait()
                pltpu.sync_copy(rows1, acc.at[dst_v.at[j + 1]], add=True)
                return carry

            lax.fori_loop(0, _ISLAB // 2 - 1, chunk_body, 0)
            j = _ISLAB - 2
            pltpu.async_copy(h_hbm.at[src_v.at[j + 1]], rows1, sem1)
            pltpu.make_async_copy(h_hbm.at[src_v.at[j]], rows0, sem0).wait()
            pltpu.sync_copy(rows0, acc.at[dst_v.at[j]], add=True)
            pltpu.make_async_copy(h_hbm.at[src_v.at[j + 1]], rows1, sem1).wait()
            pltpu.sync_copy(rows1, acc.at[dst_v.at[j + 1]], add=True)
        plsc.subcore_barrier()
        pltpu.sync_copy(acc.at[pl.ds(base, _RPT)], out_hbm.at[c, pl.ds(base, _RPT)])

    return body(h, srcp, dstp, init)


def _norm_rows(y):
    norm = jnp.sqrt(jnp.sum(y * y, axis=1, keepdims=True))
    return y / jnp.maximum(norm, 1e-12)


def _tc_norm(x):
    def body(x_ref, o_ref):
        o_ref[...] = _norm_rows(x_ref[...])

    return pl.pallas_call(
        body,
        grid=(_N_PAD // _BLK,),
        in_specs=[pl.BlockSpec((_BLK, _D), lambda i: (i, 0))],
        out_specs=pl.BlockSpec((_BLK, _D), lambda i: (i, 0)),
        out_shape=jax.ShapeDtypeStruct((_N_PAD, _D), jnp.float32),
    )(x)


def _tc_combine(p):
    def body(p_ref, o_ref):
        o_ref[...] = _norm_rows(p_ref[0] + p_ref[1])

    return pl.pallas_call(
        body,
        grid=(_N_PAD // _BLK,),
        in_specs=[pl.BlockSpec((_NC, _BLK, _D), lambda i: (0, i, 0))],
        out_specs=pl.BlockSpec((_BLK, _D), lambda i: (i, 0)),
        out_shape=jax.ShapeDtypeStruct((_N_PAD, _D), jnp.float32),
    )(p)


def kernel(x, edge_index):
    dst = edge_index[0]
    src = edge_index[1]
    pad = _EPT_PAD - _EPT
    srcp = jnp.concatenate(
        [src.reshape(_NW, _EPT),
         jnp.full((_NW, pad), _N, jnp.int32)], axis=1).reshape(
        _NW, _CHUNKS, _CHUNK)
    dummy_dst = ((jnp.arange(pad, dtype=jnp.int32)[None, :] * 41
                  + 313 * jnp.arange(_NW, dtype=jnp.int32)[:, None]) % _N)
    dstp = jnp.concatenate(
        [dst.reshape(_NW, _EPT), dummy_dst], axis=1).reshape(
        _NW, _CHUNKS, _CHUNK)

    noise_key = jax.random.key(42)
    zeros_half = jnp.zeros((1, _N_PAD, _D), jnp.float32)
    x_ext = jnp.pad(x, ((0, _N_PAD - _N), (0, 0)))
    outs = [_tc_norm(x_ext)]
    for k in range(_NUM_HOPS):
        noise = jax.random.normal(
            jax.random.fold_in(noise_key, k), (_N, _D), jnp.float32) * _NOISE_SCALE
        init = jnp.concatenate(
            [jnp.pad(noise, ((0, _N_PAD - _N), (0, 0)))[None], zeros_half], axis=0)
        p = _sc_agg(outs[k], srcp, dstp, init)
        outs.append(_tc_combine(p))
    return jnp.stack([o[:_N] for o in outs])


# speedup vs baseline: 1.4195x; 1.2498x over previous
import functools

import jax
import jax.numpy as jnp
from jax import lax
from jax.experimental import pallas as pl
from jax.experimental.pallas import tpu as pltpu
from jax.experimental.pallas import tpu_sc as plsc

_N = 10000
_E = 320000
_D = 128
_NUM_HOPS = 3
_NOISE_SCALE = 0.1

_NC = 2
_NS = 16
_NW = _NC * _NS
_CHUNK = 128
_EPT = _E // _NW
_CHUNKS = -(-_EPT // _CHUNK)
_EPT_PAD = _CHUNKS * _CHUNK
_N_PAD = 10112
_RPT = _N_PAD // _NS

_ISLAB = _CHUNKS // 2

_BLK = _N_PAD // 16


def _sc_agg(h, srcp, dstp, init):
    mesh = plsc.VectorSubcoreMesh(core_axis_name="c", subcore_axis_name="s")

    @functools.partial(
        pl.kernel,
        out_type=jax.ShapeDtypeStruct((_NC, _N_PAD, _D), jnp.float32),
        mesh=mesh,
        scratch_types=[
            pltpu.VMEM_SHARED((_N_PAD, _D), jnp.float32),
            pltpu.VMEM((_CHUNKS, _CHUNK), jnp.int32),
            pltpu.VMEM((_CHUNKS, _CHUNK), jnp.int32),
            pltpu.VMEM((_CHUNK, _D), jnp.float32),
            pltpu.SemaphoreType.DMA,
        ],
    )
    def body(h_hbm, src_hbm, dst_hbm, init_hbm, out_hbm, acc, src_v, dst_v,
             rows0, sem0):
        c = lax.axis_index("c")
        s = lax.axis_index("s")
        w = s * _NC + c
        base = s * _RPT
        pltpu.sync_copy(init_hbm.at[c, pl.ds(base, _RPT)], acc.at[pl.ds(base, _RPT)])
        plsc.subcore_barrier()

        pltpu.sync_copy(src_hbm.at[w], src_v)
        pltpu.sync_copy(dst_hbm.at[w], dst_v)

        def chunk_body(j, carry):
            pltpu.async_copy(h_hbm.at[src_v.at[j]], rows0, sem0).wait()
            pltpu.sync_copy(rows0, acc.at[dst_v.at[j]], add=True)
            return carry

        lax.fori_loop(0, _CHUNKS, chunk_body, 0)
        plsc.subcore_barrier()
        pltpu.sync_copy(acc.at[pl.ds(base, _RPT)], out_hbm.at[c, pl.ds(base, _RPT)])

    return body(h, srcp, dstp, init)


def _norm_rows(y):
    norm = jnp.sqrt(jnp.sum(y * y, axis=1, keepdims=True))
    return y / jnp.maximum(norm, 1e-12)


def _tc_norm(x):
    def body(x_ref, o_ref):
        o_ref[...] = _norm_rows(x_ref[...])

    return pl.pallas_call(
        body,
        grid=(_N_PAD // _BLK,),
        in_specs=[pl.BlockSpec((_BLK, _D), lambda i: (i, 0))],
        out_specs=pl.BlockSpec((_BLK, _D), lambda i: (i, 0)),
        out_shape=jax.ShapeDtypeStruct((_N_PAD, _D), jnp.float32),
    )(x)


def _tc_combine(p):
    def body(p_ref, o_ref):
        o_ref[...] = _norm_rows(p_ref[0] + p_ref[1])

    return pl.pallas_call(
        body,
        grid=(_N_PAD // _BLK,),
        in_specs=[pl.BlockSpec((_NC, _BLK, _D), lambda i: (0, i, 0))],
        out_specs=pl.BlockSpec((_BLK, _D), lambda i: (i, 0)),
        out_shape=jax.ShapeDtypeStruct((_N_PAD, _D), jnp.float32),
    )(p)


def kernel(x, edge_index):
    dst = edge_index[0]
    src = edge_index[1]
    pad = _EPT_PAD - _EPT
    srcp = jnp.concatenate(
        [src.reshape(_NW, _EPT),
         jnp.full((_NW, pad), _N, jnp.int32)], axis=1).reshape(
        _NW, _CHUNKS, _CHUNK)
    dummy_dst = ((jnp.arange(pad, dtype=jnp.int32)[None, :] * 41
                  + 313 * jnp.arange(_NW, dtype=jnp.int32)[:, None]) % _N)
    dstp = jnp.concatenate(
        [dst.reshape(_NW, _EPT), dummy_dst], axis=1).reshape(
        _NW, _CHUNKS, _CHUNK)

    noise_key = jax.random.key(42)
    zeros_half = jnp.zeros((1, _N_PAD, _D), jnp.float32)
    x_ext = jnp.pad(x, ((0, _N_PAD - _N), (0, 0)))
    outs = [_tc_norm(x_ext)]
    for k in range(_NUM_HOPS):
        noise = jax.random.normal(
            jax.random.fold_in(noise_key, k), (_N, _D), jnp.float32) * _NOISE_SCALE
        init = jnp.concatenate(
            [jnp.pad(noise, ((0, _N_PAD - _N), (0, 0)))[None], zeros_half], axis=0)
        p = _sc_agg(outs[k], srcp, dstp, init)
        outs.append(_tc_combine(p))
    return jnp.stack([o[:_N] for o in outs])
